# baseline (speedup 1.0000x reference)
import functools

import jax
import jax.numpy as jnp
from jax import lax
from jax.experimental import pallas as pl
from jax.experimental.pallas import tpu as pltpu
from jax.experimental.pallas import tpu_sc as plsc

_VOCAB = 1000000
_DIM = 32
_BATCH = 4096
_HIST = 200
_SCALE = float(_DIM) ** 0.5

_NW = 32
_BW = _BATCH // _NW


def _sc_body(ids_hbm, table_hbm, out_hbm,
             idx_all, g0, g1, t0, t1, sg0, sg1, ss0, ss1):
    wid = lax.axis_index("s") * 2 + lax.axis_index("c")
    wb = wid * _BW

    gbuf = (g0, g1)
    tbuf = (t0, t1)
    gsem = (sg0, sg1)
    ssem = (ss0, ss1)

    pltpu.sync_copy(ids_hbm.at[:, pl.ds(wb, _BW)], idx_all)

    def start_gather(h, b):
        pltpu.async_copy(table_hbm.at[idx_all.at[h]], gbuf[b], gsem[b])

    def wait_gather(b):
        pltpu.make_async_copy(
            table_hbm.at[idx_all.at[0]], gbuf[b], gsem[b]).wait()

    def start_scatter(h, b):
        pltpu.async_copy(
            tbuf[b], out_hbm.at[h, :, pl.ds(wb, _BW)], ssem[b])

    def wait_scatter(b):
        pltpu.make_async_copy(
            tbuf[b], out_hbm.at[0, :, pl.ds(wb, _BW)], ssem[b]).wait()

    row_ids = [
        jax.lax.iota(jnp.int32, 16) + (r * 16) for r in range(_BW // 16)
    ]

    def transpose_scale(b):
        src = gbuf[b]
        dst = tbuf[b]
        for d in range(_DIM):
            dcol = jnp.full((16,), d, jnp.int32)
            for r in range(_BW // 16):
                v = plsc.load_gather(src, [row_ids[r], dcol])
                dst[d, pl.ds(r * 16, 16)] = v * _SCALE

    start_gather(0, 0)
    start_gather(1, 1)

    for b in (0, 1):
        wait_gather(b)
        transpose_scale(b)
        start_scatter(b, b)
        start_gather(2 + b, b)

    def round_body(r, _):
        for b in (0, 1):
            h = 2 * r + b
            wait_gather(b)
            wait_scatter(b)
            transpose_scale(b)
            start_scatter(h, b)

            @pl.when(h + 2 < _HIST)
            def _():
                start_gather(h + 2, b)

        return 0

    lax.fori_loop(1, _HIST // 2, round_body, 0)

    wait_scatter(0)
    wait_scatter(1)


@jax.jit
def _embed(ids_t, table):
    mesh = plsc.VectorSubcoreMesh(core_axis_name="c", subcore_axis_name="s")
    run = functools.partial(
        pl.kernel,
        mesh=mesh,
        out_type=jax.ShapeDtypeStruct((_HIST, _DIM, _BATCH), jnp.float32),
        scratch_types=[
            pltpu.VMEM((_HIST, _BW), jnp.int32),
            pltpu.VMEM((_BW, _DIM), jnp.float32),
            pltpu.VMEM((_BW, _DIM), jnp.float32),
            pltpu.VMEM((_DIM, _BW), jnp.float32),
            pltpu.VMEM((_DIM, _BW), jnp.float32),
            pltpu.SemaphoreType.DMA,
            pltpu.SemaphoreType.DMA,
            pltpu.SemaphoreType.DMA,
            pltpu.SemaphoreType.DMA,
        ],
        compiler_params=pltpu.CompilerParams(use_tc_tiling_on_sc=False, needs_layout_passes=False),
    )(_sc_body)
    return run(ids_t, table)


def kernel(input_ids, table):
    ids_t = input_ids.astype(jnp.int32).T
    out_phys = _embed(ids_t, table)
    return jnp.transpose(out_phys, (2, 0, 1))


# speedup vs baseline: 1.7992x; 1.7992x over previous
import functools

import jax
import jax.numpy as jnp
from jax import lax
from jax.experimental import pallas as pl
from jax.experimental.pallas import tpu as pltpu
from jax.experimental.pallas import tpu_sc as plsc

_VOCAB = 1000000
_DIM = 32
_BATCH = 4096
_HIST = 200
_SCALE = float(_DIM) ** 0.5

_NW = 32
_BW = _BATCH // _NW


def _sc_body(ids_hbm, table_hbm, out_hbm,
             idx_all, g0, g1, t0, t1, sg0, sg1, ss0, ss1):
    wid = lax.axis_index("s") * 2 + lax.axis_index("c")
    wb = wid * _BW

    gbuf = (g0, g1)
    tbuf = (t0, t1)
    gsem = (sg0, sg1)
    ssem = (ss0, ss1)

    pltpu.sync_copy(ids_hbm.at[:, pl.ds(wb, _BW)], idx_all)

    def start_gather(h, b):
        pltpu.async_copy(table_hbm.at[idx_all.at[h]], gbuf[b], gsem[b])

    def wait_gather(b):
        pltpu.make_async_copy(
            table_hbm.at[idx_all.at[0]], gbuf[b], gsem[b]).wait()

    def start_scatter(h, b):
        pltpu.async_copy(
            tbuf[b], out_hbm.at[h, :, pl.ds(wb, _BW)], ssem[b])

    def wait_scatter(b):
        pltpu.make_async_copy(
            tbuf[b], out_hbm.at[0, :, pl.ds(wb, _BW)], ssem[b]).wait()

    lane = jax.lax.iota(jnp.int32, 16)
    perms = [jnp.bitwise_and(lane + k, 15) for k in range(16)]

    def transpose_scale(b):
        src = gbuf[b]
        dst = tbuf[b]
        def rg_body(rg, _):
            rows = lane + rg * 16
            for cg in range(_DIM // 16):
                for k in range(16):
                    cols = perms[k] + (cg * 16)
                    v = plsc.load_gather(src, [rows, cols])
                    plsc.store_scatter(dst, [cols, rows], v * _SCALE)
            return 0

        lax.fori_loop(0, _BW // 16, rg_body, 0)

    start_gather(0, 0)
    start_gather(1, 1)

    for b in (0, 1):
        wait_gather(b)
        transpose_scale(b)
        start_scatter(b, b)
        start_gather(2 + b, b)

    def round_body(r, _):
        for b in (0, 1):
            h = 2 * r + b
            wait_gather(b)
            wait_scatter(b)
            transpose_scale(b)
            start_scatter(h, b)

            @pl.when(h + 2 < _HIST)
            def _():
                start_gather(h + 2, b)

        return 0

    lax.fori_loop(1, _HIST // 2, round_body, 0)

    wait_scatter(0)
    wait_scatter(1)


@jax.jit
def _embed(ids_t, table):
    mesh = plsc.VectorSubcoreMesh(core_axis_name="c", subcore_axis_name="s")
    run = functools.partial(
        pl.kernel,
        mesh=mesh,
        out_type=jax.ShapeDtypeStruct((_HIST, _DIM, _BATCH), jnp.float32),
        scratch_types=[
            pltpu.VMEM((_HIST, _BW), jnp.int32),
            pltpu.VMEM((_BW, _DIM), jnp.float32),
            pltpu.VMEM((_BW, _DIM), jnp.float32),
            pltpu.VMEM((_DIM, _BW), jnp.float32),
            pltpu.VMEM((_DIM, _BW), jnp.float32),
            pltpu.SemaphoreType.DMA,
            pltpu.SemaphoreType.DMA,
            pltpu.SemaphoreType.DMA,
            pltpu.SemaphoreType.DMA,
        ],
        compiler_params=pltpu.CompilerParams(use_tc_tiling_on_sc=False, needs_layout_passes=False),
    )(_sc_body)
    return run(ids_t, table)


def kernel(input_ids, table):
    ids_t = input_ids.astype(jnp.int32).T
    out_phys = _embed(ids_t, table)
    return jnp.transpose(out_phys, (2, 0, 1))


# speedup vs baseline: 2.1673x; 1.2046x over previous
import functools

import jax
import jax.numpy as jnp
from jax import lax
from jax.experimental import pallas as pl
from jax.experimental.pallas import tpu as pltpu
from jax.experimental.pallas import tpu_sc as plsc

_VOCAB = 1000000
_DIM = 32
_BATCH = 4096
_HIST = 200
_SCALE = float(_DIM) ** 0.5

_NW = 32
_BW = _BATCH // _NW
_TC_FULL = _VOCAB // 128
_TC_TAIL = _VOCAB - _TC_FULL * 128
_TC_LOOP = _TC_FULL // _NW
_TC_REM = _TC_FULL - _TC_LOOP * _NW

_LANE16 = None


def _diag_perms():
    lane = jax.lax.iota(jnp.int32, 16)
    return lane, [jnp.bitwise_and(lane + k, 15) for k in range(16)]


def _fmt_body(tt_hbm, tail_hbm, out_hbm, i0, i1, o0, o1, si0, si1, so0, so1):
    wid = lax.axis_index("s") * 2 + lax.axis_index("c")

    ibuf = (i0, i1)
    obuf = (o0, o1)
    isem = (si0, si1)
    osem = (so0, so1)

    lane, perms = _diag_perms()

    def start_in(c, b):
        pltpu.async_copy(
            tt_hbm.at[:, pl.ds(c * 128, 128)], ibuf[b], isem[b])

    def wait_in(b):
        pltpu.make_async_copy(
            tt_hbm.at[:, pl.ds(0, 128)], ibuf[b], isem[b]).wait()

    def start_out(c, b):
        pltpu.async_copy(
            obuf[b], out_hbm.at[pl.ds(c * 4096, 4096)], osem[b])

    def wait_out(b):
        pltpu.make_async_copy(
            obuf[b], out_hbm.at[pl.ds(0, 4096)], osem[b]).wait()

    def transpose_scale(b, ncg):
        src = ibuf[b]
        dst = obuf[b]

        def rg_body(rg, _):
            rows = lane + rg * 16
            for cg in range(ncg):
                for k in range(16):
                    cols = perms[k] + cg * 16
                    v = plsc.load_gather(src, [rows, cols])
                    plsc.store_scatter(dst, [cols * 32 + rows], v * _SCALE)
            return 0

        lax.fori_loop(0, 2, rg_body, 0)

    start_in(wid, 0)
    start_in(32 + wid, 1)

    for b in (0, 1):
        wait_in(b)
        transpose_scale(b, 8)
        start_out(b * 32 + wid, b)
        start_in((b + 2) * 32 + wid, b)

    def loop_body(k, _):
        for b in (0, 1):
            c = (2 * k + b) * 32 + wid
            wait_in(b)
            wait_out(b)
            transpose_scale(b, 8)
            start_out(c, b)

            @pl.when(c + 64 < _TC_LOOP * _NW)
            def _():
                start_in(c + 64, b)

        return 0

    lax.fori_loop(1, _TC_LOOP // 2, loop_body, 0)
    wait_out(0)
    wait_out(1)

    @pl.when(wid < _TC_REM)
    def _():
        c = _TC_LOOP * _NW + wid
        start_in(c, 0)
        wait_in(0)
        transpose_scale(0, 8)
        start_out(c, 0)
        wait_out(0)

    @pl.when(wid == _TC_REM)
    def _():
        pltpu.sync_copy(tail_hbm, o1.at[pl.ds(0, _TC_TAIL * _DIM)])

        def tail_rg(rg, _):
            rows = lane + rg * 16
            for cg in range(_TC_TAIL // 16):
                for k in range(16):
                    cols = perms[k] + cg * 16
                    v = plsc.load_gather(o1, [rows * _TC_TAIL + cols])
                    plsc.store_scatter(o0, [cols * 32 + rows], v * _SCALE)
            return 0

        lax.fori_loop(0, 2, tail_rg, 0)
        pltpu.sync_copy(
            o0.at[pl.ds(0, _TC_TAIL * _DIM)],
            out_hbm.at[pl.ds(_TC_FULL * 4096, _TC_TAIL * _DIM)])


def _lookup_body(ids_hbm, table_hbm, out_hbm,
                 idx_all, g0, g1, t0, t1, sg0, sg1, ss0, ss1):
    wid = lax.axis_index("s") * 2 + lax.axis_index("c")
    wb = wid * _BW

    gbuf = (g0, g1)
    tbuf = (t0, t1)
    gsem = (sg0, sg1)
    ssem = (ss0, ss1)

    lane, perms = _diag_perms()

    pltpu.sync_copy(ids_hbm.at[:, pl.ds(wb, _BW)], idx_all)

    def start_gather(h, b):
        pltpu.async_copy(table_hbm.at[idx_all.at[h]], gbuf[b], gsem[b])

    def wait_gather(b):
        pltpu.make_async_copy(
            table_hbm.at[idx_all.at[0]], gbuf[b], gsem[b]).wait()

    def start_scatter(h, b):
        pltpu.async_copy(
            tbuf[b], out_hbm.at[h, :, wid, :, :], ssem[b])

    def wait_scatter(b):
        pltpu.make_async_copy(
            tbuf[b], out_hbm.at[0, :, wid, :, :], ssem[b]).wait()

    def transpose(b):
        src = gbuf[b]
        dst = tbuf[b]

        def rg_body(rg, _):
            rows = lane + rg * 16
            for cg in range(_DIM // 16):
                for k in range(16):
                    cols = perms[k] + cg * 16
                    v = plsc.load_gather(src, [rows, cols])
                    plsc.store_scatter(
                        dst,
                        [jax.lax.shift_right_logical(cols, 3),
                         jnp.bitwise_and(cols, 7),
                         rows],
                        v)
            return 0

        lax.fori_loop(0, _BW // 16, rg_body, 0)

    start_gather(0, 0)
    start_gather(1, 1)

    for b in (0, 1):
        wait_gather(b)
        transpose(b)
        start_scatter(b, b)
        start_gather(2 + b, b)

    def round_body(r, _):
        for b in (0, 1):
            h = 2 * r + b
            wait_gather(b)
            wait_scatter(b)
            transpose(b)
            start_scatter(h, b)

            @pl.when(h + 2 < _HIST)
            def _():
                start_gather(h + 2, b)

        return 0

    lax.fori_loop(1, _HIST // 2, round_body, 0)
    wait_scatter(0)
    wait_scatter(1)


@jax.jit
def _embed(ids_t, table_t):
    mesh = plsc.VectorSubcoreMesh(core_axis_name="c", subcore_axis_name="s")

    fmt = functools.partial(
        pl.kernel,
        mesh=mesh,
        out_type=jax.ShapeDtypeStruct((_VOCAB * _DIM,), jnp.float32),
        scratch_types=[
            pltpu.VMEM((_DIM, 128), jnp.float32),
            pltpu.VMEM((_DIM, 128), jnp.float32),
            pltpu.VMEM((4096,), jnp.float32),
            pltpu.VMEM((4096,), jnp.float32),
            pltpu.SemaphoreType.DMA,
            pltpu.SemaphoreType.DMA,
            pltpu.SemaphoreType.DMA,
            pltpu.SemaphoreType.DMA,
        ],
        compiler_params=pltpu.CompilerParams(
            use_tc_tiling_on_sc=True, needs_layout_passes=False),
    )(_fmt_body)
    tail_flat = table_t[:, _TC_FULL * 128:].reshape(-1)
    table_rm = fmt(table_t, tail_flat).reshape(_VOCAB, _DIM)

    lookup = functools.partial(
        pl.kernel,
        mesh=mesh,
        out_type=jax.ShapeDtypeStruct(
            (_HIST, _DIM // 8, _BATCH // 128, 8, 128), jnp.float32),
        scratch_types=[
            pltpu.VMEM((_HIST, _BW), jnp.int32),
            pltpu.VMEM((_BW, _DIM), jnp.float32),
            pltpu.VMEM((_BW, _DIM), jnp.float32),
            pltpu.VMEM((_DIM // 8, 8, _BW), jnp.float32),
            pltpu.VMEM((_DIM // 8, 8, _BW), jnp.float32),
            pltpu.SemaphoreType.DMA,
            pltpu.SemaphoreType.DMA,
            pltpu.SemaphoreType.DMA,
            pltpu.SemaphoreType.DMA,
        ],
        compiler_params=pltpu.CompilerParams(
            use_tc_tiling_on_sc=False, needs_layout_passes=False),
    )(_lookup_body)
    out5 = lookup(ids_t, table_rm)

    out = jnp.transpose(out5, (2, 4, 0, 1, 3))
    return out.reshape(_BATCH, _HIST, _DIM)


def kernel(input_ids, table):
    return _embed(input_ids.astype(jnp.int32).T, table.T)
